# baseline (speedup 1.0000x reference)
import functools

import jax
import jax.numpy as jnp
from jax import lax
from jax.experimental import pallas as pl
from jax.experimental.pallas import tpu as pltpu
from jax.experimental.pallas import tpu_sc as plsc

N = 10000
E = 160000
D = 256
HALF = 128
NPAD = 10240
EPAD = 163840
BLK = 128
NBLK = EPAD // BLK
ROWS_PER_TILE = NPAD // 16

_mesh = plsc.VectorSubcoreMesh(core_axis_name="c", subcore_axis_name="s")


@functools.partial(
    pl.kernel, mesh=_mesh,
    out_type=jax.ShapeDtypeStruct((2, NPAD), jnp.float32),
    scratch_types=[
        pltpu.VMEM((BLK,), jnp.float32),
        pltpu.VMEM((NBLK // 32, BLK), jnp.int32),
        pltpu.VMEM_SHARED((NPAD,), jnp.float32),
    ],
)
def _sc_deg(dst_hbm, zeros_hbm, ones_hbm, out_hbm, ones_v, dst_all, acc):
    c = lax.axis_index("c")
    s = lax.axis_index("s")
    r0 = pl.multiple_of(s * ROWS_PER_TILE, ROWS_PER_TILE)
    pltpu.sync_copy(zeros_hbm.at[pl.ds(r0, ROWS_PER_TILE)],
                    acc.at[pl.ds(r0, ROWS_PER_TILE)])
    pltpu.sync_copy(ones_hbm, ones_v)
    w = c * 16 + s
    nb = NBLK // 32
    pltpu.sync_copy(dst_hbm.at[pl.ds(w * nb, nb)], dst_all)
    plsc.subcore_barrier()

    def body(i, carry):
        pltpu.sync_copy(ones_v, acc.at[dst_all.at[i]], add=True)
        return carry

    lax.fori_loop(0, nb, body, 0)
    plsc.subcore_barrier()
    pltpu.sync_copy(acc.at[pl.ds(r0, ROWS_PER_TILE)],
                    out_hbm.at[c, pl.ds(r0, ROWS_PER_TILE)])


NRING = 4
NRING_AGG = 2
HB = NBLK // 32

@functools.partial(
    pl.kernel, mesh=_mesh,
    out_type=jax.ShapeDtypeStruct((2, NPAD, HALF), jnp.float32),
    scratch_types=[
        pltpu.VMEM((HB, BLK), jnp.int32),
        pltpu.VMEM((HB, BLK), jnp.int32),
    ] + [pltpu.VMEM((BLK, HALF), jnp.float32) for _ in range(NRING_AGG)]
      + [pltpu.VMEM_SHARED((NPAD, HALF), jnp.float32)]
      + [pltpu.SemaphoreType.DMA for _ in range(2 * NRING_AGG)],
)
def _sc_agg(hs0_hbm, hs1_hbm, src_hbm, dst_hbm, z2d_hbm, out_hbm,
            src_all, dst_all, r0_v, r1_v, acc, gsem0, gsem1, ssem0, ssem1):
    rows = (r0_v, r1_v)
    gsems = (gsem0, gsem1)
    ssems = (ssem0, ssem1)
    c = lax.axis_index("c")
    s = lax.axis_index("s")
    for k in range(ROWS_PER_TILE // BLK):
        r = pl.multiple_of(s * ROWS_PER_TILE + k * BLK, BLK)
        pltpu.sync_copy(z2d_hbm, acc.at[pl.ds(r, BLK)])
    plsc.subcore_barrier()

    def run(table):
        def wait_s(b):
            pltpu.make_async_copy(
                rows[b], acc.at[dst_all.at[0]], ssems[b]).wait()

        for h in range(2):
            base = pl.multiple_of(s * (2 * HB) + h * HB, HB)
            pltpu.sync_copy(src_hbm.at[pl.ds(base, HB)], src_all)
            pltpu.sync_copy(dst_hbm.at[pl.ds(base, HB)], dst_all)
            pltpu.async_copy(table.at[src_all.at[0]], rows[0], gsems[0])

            def body(k, carry):
                for b in range(2):
                    j = 2 * k + b
                    pltpu.make_async_copy(
                        table.at[src_all.at[b]], rows[b], gsems[b]).wait()
                    pltpu.async_copy(
                        rows[b], acc.at[dst_all.at[j]], ssems[b], add=True)
                    if b == 0:
                        @pl.when(k >= 1)
                        def _():
                            wait_s(1)
                        pltpu.async_copy(
                            table.at[src_all.at[j + 1]], rows[1], gsems[1])
                    else:
                        wait_s(0)

                        @pl.when(j + 1 < HB)
                        def _():
                            pltpu.async_copy(
                                table.at[src_all.at[j + 1]], rows[0],
                                gsems[0])
                return carry

            lax.fori_loop(0, HB // 2, body, 0)
            wait_s(1)

    @pl.when(c == 0)
    def _():
        run(hs0_hbm)

    @pl.when(c == 1)
    def _():
        run(hs1_hbm)

    plsc.subcore_barrier()
    for k in range(ROWS_PER_TILE // BLK):
        r = pl.multiple_of(s * ROWS_PER_TILE + k * BLK, BLK)
        pltpu.sync_copy(acc.at[pl.ds(r, BLK)], out_hbm.at[c, pl.ds(r, BLK)])


@functools.partial(
    pl.kernel, mesh=_mesh,
    out_type=jax.ShapeDtypeStruct((2, NPAD), jnp.float32),
    scratch_types=[
        pltpu.VMEM((NBLK // 32, BLK), jnp.int32),
        pltpu.VMEM((NBLK // 32, BLK), jnp.int32),
    ] + [pltpu.VMEM((BLK,), jnp.float32) for _ in range(NRING)]
      + [pltpu.VMEM_SHARED((NPAD,), jnp.float32)]
      + [pltpu.SemaphoreType.DMA for _ in range(NRING)],
)
def _sc_agg2(h2s_hbm, src_hbm, dst_hbm, zeros_hbm, out_hbm,
             src_all, dst_all, m0_v, m1_v, m2_v, m3_v, acc,
             sem0, sem1, sem2, sem3):
    msgs = (m0_v, m1_v, m2_v, m3_v)
    sems = (sem0, sem1, sem2, sem3)
    c = lax.axis_index("c")
    s = lax.axis_index("s")
    r0 = pl.multiple_of(s * ROWS_PER_TILE, ROWS_PER_TILE)
    pltpu.sync_copy(zeros_hbm.at[pl.ds(r0, ROWS_PER_TILE)],
                    acc.at[pl.ds(r0, ROWS_PER_TILE)])
    w = c * 16 + s
    nb = NBLK // 32
    pltpu.sync_copy(src_hbm.at[pl.ds(w * nb, nb)], src_all)
    pltpu.sync_copy(dst_hbm.at[pl.ds(w * nb, nb)], dst_all)
    plsc.subcore_barrier()
    for b in range(NRING):
        pltpu.async_copy(h2s_hbm.at[src_all.at[b]], msgs[b], sems[b])

    def body(k, carry):
        for b in range(NRING):
            blk = k * NRING + b
            pltpu.make_async_copy(
                h2s_hbm.at[src_all.at[b]], msgs[b], sems[b]).wait()
            pltpu.sync_copy(msgs[b], acc.at[dst_all.at[blk]], add=True)

            @pl.when(blk + NRING < nb)
            def _():
                pltpu.async_copy(
                    h2s_hbm.at[src_all.at[blk + NRING]], msgs[b], sems[b])
        return carry

    lax.fori_loop(0, nb // NRING, body, 0)
    plsc.subcore_barrier()
    pltpu.sync_copy(acc.at[pl.ds(r0, ROWS_PER_TILE)],
                    out_hbm.at[c, pl.ds(r0, ROWS_PER_TILE)])


def _tc1_body(x_ref, w1_ref, deg_ref, hs0_ref, hs1_ref, dis_ref):
    d2 = deg_ref[...]
    dis = lax.rsqrt(d2[0] + d2[1] + 1.0)
    h = jnp.dot(x_ref[...], w1_ref[...], preferred_element_type=jnp.float32)
    hs = h * dis
    hs0_ref[...] = hs[:, :HALF]
    hs1_ref[...] = hs[:, HALF:]
    dis_ref[...] = dis


def _tc2_body(agg0_ref, agg1_ref, hs0_ref, hs1_ref, dis_ref,
              b1a_ref, b1b_ref, w2a_ref, w2b_ref, h2s_ref):
    dis = dis_ref[...]
    o0 = jnp.maximum(dis * (agg0_ref[...][0] + hs0_ref[...]) + b1a_ref[...], 0.0)
    o1 = jnp.maximum(dis * (agg1_ref[...][0] + hs1_ref[...]) + b1b_ref[...], 0.0)
    h2 = jnp.sum(o0 * w2a_ref[...] + o1 * w2b_ref[...], axis=1, keepdims=True)
    h2s_ref[...] = h2 * dis


def _tc3_body(agg2_ref, h2s_ref, dis_ref, b2_ref, out_ref):
    a = agg2_ref[...]
    out_ref[...] = dis_ref[...] * (a[0] + a[1] + h2s_ref[...]) + b2_ref[...]


RB = 1000


def kernel(x, edge_index, W1, b1, W2, b2):
    ei = edge_index.astype(jnp.int32)
    src_p = jnp.concatenate(
        [ei[0], jnp.zeros((EPAD - E,), jnp.int32)]).reshape(NBLK, BLK)
    dst_p = jnp.concatenate(
        [ei[1], jnp.full((EPAD - E,), N, jnp.int32)]).reshape(NBLK, BLK)
    zeros_1d = jnp.zeros((NPAD,), jnp.float32)
    ones_blk = jnp.ones((BLK,), jnp.float32)
    zeros_2d = jnp.zeros((BLK, HALF), jnp.float32)

    deg2 = _sc_deg(dst_p, zeros_1d, ones_blk)

    grid = (N // RB,)
    hs0, hs1, dis = pl.pallas_call(
        _tc1_body,
        grid=grid,
        in_specs=[
            pl.BlockSpec((RB, D), lambda i: (i, 0)),
            pl.BlockSpec((D, D), lambda i: (0, 0)),
            pl.BlockSpec((2, RB, 1), lambda i: (0, i, 0)),
        ],
        out_specs=[
            pl.BlockSpec((RB, HALF), lambda i: (i, 0)),
            pl.BlockSpec((RB, HALF), lambda i: (i, 0)),
            pl.BlockSpec((RB, 1), lambda i: (i, 0)),
        ],
        out_shape=[
            jax.ShapeDtypeStruct((N, HALF), jnp.float32),
            jax.ShapeDtypeStruct((N, HALF), jnp.float32),
            jax.ShapeDtypeStruct((N, 1), jnp.float32),
        ],
    )(x, W1, deg2.reshape(2, NPAD, 1))

    agg = _sc_agg(hs0, hs1, src_p, dst_p, zeros_2d)

    b1r = b1.reshape(1, D)
    w2r = W2.reshape(1, D)
    h2s = pl.pallas_call(
        _tc2_body,
        grid=grid,
        in_specs=[
            pl.BlockSpec((1, RB, HALF), lambda i: (0, i, 0)),
            pl.BlockSpec((1, RB, HALF), lambda i: (1, i, 0)),
            pl.BlockSpec((RB, HALF), lambda i: (i, 0)),
            pl.BlockSpec((RB, HALF), lambda i: (i, 0)),
            pl.BlockSpec((RB, 1), lambda i: (i, 0)),
            pl.BlockSpec((1, HALF), lambda i: (0, 0)),
            pl.BlockSpec((1, HALF), lambda i: (0, 1)),
            pl.BlockSpec((1, HALF), lambda i: (0, 0)),
            pl.BlockSpec((1, HALF), lambda i: (0, 1)),
        ],
        out_specs=pl.BlockSpec((RB, 1), lambda i: (i, 0)),
        out_shape=jax.ShapeDtypeStruct((N, 1), jnp.float32),
    )(agg, agg, hs0, hs1, dis, b1r, b1r, w2r, w2r)

    agg2 = _sc_agg2(h2s.reshape(N), src_p, dst_p, zeros_1d)

    out = pl.pallas_call(
        _tc3_body,
        grid=grid,
        in_specs=[
            pl.BlockSpec((2, RB, 1), lambda i: (0, i, 0)),
            pl.BlockSpec((RB, 1), lambda i: (i, 0)),
            pl.BlockSpec((RB, 1), lambda i: (i, 0)),
            pl.BlockSpec((1, 1), lambda i: (0, 0)),
        ],
        out_specs=pl.BlockSpec((RB, 1), lambda i: (i, 0)),
        out_shape=jax.ShapeDtypeStruct((N, 1), jnp.float32),
    )(agg2.reshape(2, NPAD, 1), h2s, dis, b2.reshape(1, 1))
    return out


# speedup vs baseline: 1.1281x; 1.1281x over previous
import functools

import jax
import jax.numpy as jnp
from jax import lax
from jax.experimental import pallas as pl
from jax.experimental.pallas import tpu as pltpu
from jax.experimental.pallas import tpu_sc as plsc

N = 10000
E = 160000
D = 256
HALF = 128
NPAD = 10240
EPAD = 163840
BLK = 128
NBLK = EPAD // BLK
ROWS_PER_TILE = NPAD // 16

_mesh = plsc.VectorSubcoreMesh(core_axis_name="c", subcore_axis_name="s")


@functools.partial(
    pl.kernel, mesh=_mesh,
    out_type=jax.ShapeDtypeStruct((2, NPAD), jnp.float32),
    scratch_types=[
        pltpu.VMEM((BLK,), jnp.float32),
        pltpu.VMEM((NBLK // 32, BLK), jnp.int32),
        pltpu.VMEM_SHARED((NPAD,), jnp.float32),
    ],
)
def _sc_deg(dst_hbm, zeros_hbm, ones_hbm, out_hbm, ones_v, dst_all, acc):
    c = lax.axis_index("c")
    s = lax.axis_index("s")
    r0 = pl.multiple_of(s * ROWS_PER_TILE, ROWS_PER_TILE)
    pltpu.sync_copy(zeros_hbm.at[pl.ds(r0, ROWS_PER_TILE)],
                    acc.at[pl.ds(r0, ROWS_PER_TILE)])
    pltpu.sync_copy(ones_hbm, ones_v)
    w = c * 16 + s
    nb = NBLK // 32
    pltpu.sync_copy(dst_hbm.at[pl.ds(w * nb, nb)], dst_all)
    plsc.subcore_barrier()

    def body(i, carry):
        pltpu.sync_copy(ones_v, acc.at[dst_all.at[i]], add=True)
        return carry

    lax.fori_loop(0, nb, body, 0)
    plsc.subcore_barrier()
    pltpu.sync_copy(acc.at[pl.ds(r0, ROWS_PER_TILE)],
                    out_hbm.at[c, pl.ds(r0, ROWS_PER_TILE)])


NRING = 4
NRING_AGG = 2
HB = NBLK // 32

@functools.partial(
    pl.kernel, mesh=_mesh,
    out_type=jax.ShapeDtypeStruct((2, NPAD, HALF), jnp.float32),
    scratch_types=[
        pltpu.VMEM((HB, BLK), jnp.int32),
        pltpu.VMEM((HB, BLK), jnp.int32),
    ] + [pltpu.VMEM((BLK, HALF), jnp.float32) for _ in range(NRING_AGG)]
      + [pltpu.VMEM_SHARED((NPAD, HALF), jnp.float32)]
      + [pltpu.SemaphoreType.DMA for _ in range(NRING_AGG)],
)
def _sc_agg(hs0_hbm, hs1_hbm, src_hbm, dst_hbm, z2d_hbm, out_hbm,
            src_all, dst_all, r0_v, r1_v, acc, sem0, sem1):
    rows = (r0_v, r1_v)
    sems = (sem0, sem1)
    c = lax.axis_index("c")
    s = lax.axis_index("s")
    for k in range(ROWS_PER_TILE // BLK):
        r = pl.multiple_of(s * ROWS_PER_TILE + k * BLK, BLK)
        pltpu.sync_copy(z2d_hbm, acc.at[pl.ds(r, BLK)])
    plsc.subcore_barrier()

    def run(table):
        for h in range(2):
            base = pl.multiple_of(s * (2 * HB) + h * HB, HB)
            pltpu.sync_copy(src_hbm.at[pl.ds(base, HB)], src_all)
            pltpu.sync_copy(dst_hbm.at[pl.ds(base, HB)], dst_all)
            for b in range(NRING_AGG):
                pltpu.async_copy(table.at[src_all.at[b]], rows[b], sems[b])

            def body(k, carry):
                for b in range(NRING_AGG):
                    j = k * NRING_AGG + b
                    pltpu.make_async_copy(
                        table.at[src_all.at[b]], rows[b], sems[b]).wait()
                    pltpu.sync_copy(rows[b], acc.at[dst_all.at[j]], add=True)

                    @pl.when(j + NRING_AGG < HB)
                    def _():
                        pltpu.async_copy(
                            table.at[src_all.at[j + NRING_AGG]],
                            rows[b], sems[b])
                return carry

            lax.fori_loop(0, HB // NRING_AGG, body, 0)

    @pl.when(c == 0)
    def _():
        run(hs0_hbm)

    @pl.when(c == 1)
    def _():
        run(hs1_hbm)

    plsc.subcore_barrier()
    for k in range(ROWS_PER_TILE // BLK):
        r = pl.multiple_of(s * ROWS_PER_TILE + k * BLK, BLK)
        pltpu.sync_copy(acc.at[pl.ds(r, BLK)], out_hbm.at[c, pl.ds(r, BLK)])


@functools.partial(
    pl.kernel, mesh=_mesh,
    out_type=jax.ShapeDtypeStruct((2, NPAD), jnp.float32),
    scratch_types=[
        pltpu.VMEM((NBLK // 32, BLK), jnp.int32),
        pltpu.VMEM((NBLK // 32, BLK), jnp.int32),
    ] + [pltpu.VMEM((BLK,), jnp.float32) for _ in range(NRING)]
      + [pltpu.VMEM_SHARED((NPAD,), jnp.float32)]
      + [pltpu.SemaphoreType.DMA for _ in range(NRING)],
)
def _sc_agg2(h2s_hbm, src_hbm, dst_hbm, zeros_hbm, out_hbm,
             src_all, dst_all, m0_v, m1_v, m2_v, m3_v, acc,
             sem0, sem1, sem2, sem3):
    msgs = (m0_v, m1_v, m2_v, m3_v)
    sems = (sem0, sem1, sem2, sem3)
    c = lax.axis_index("c")
    s = lax.axis_index("s")
    r0 = pl.multiple_of(s * ROWS_PER_TILE, ROWS_PER_TILE)
    pltpu.sync_copy(zeros_hbm.at[pl.ds(r0, ROWS_PER_TILE)],
                    acc.at[pl.ds(r0, ROWS_PER_TILE)])
    w = c * 16 + s
    nb = NBLK // 32
    pltpu.sync_copy(src_hbm.at[pl.ds(w * nb, nb)], src_all)
    pltpu.sync_copy(dst_hbm.at[pl.ds(w * nb, nb)], dst_all)
    plsc.subcore_barrier()
    for b in range(NRING):
        pltpu.async_copy(h2s_hbm.at[src_all.at[b]], msgs[b], sems[b])

    def body(k, carry):
        for b in range(NRING):
            blk = k * NRING + b
            pltpu.make_async_copy(
                h2s_hbm.at[src_all.at[b]], msgs[b], sems[b]).wait()
            pltpu.sync_copy(msgs[b], acc.at[dst_all.at[blk]], add=True)

            @pl.when(blk + NRING < nb)
            def _():
                pltpu.async_copy(
                    h2s_hbm.at[src_all.at[blk + NRING]], msgs[b], sems[b])
        return carry

    lax.fori_loop(0, nb // NRING, body, 0)
    plsc.subcore_barrier()
    pltpu.sync_copy(acc.at[pl.ds(r0, ROWS_PER_TILE)],
                    out_hbm.at[c, pl.ds(r0, ROWS_PER_TILE)])


def _tc1a_body(x_ref, w1_ref, h_ref):
    h_ref[...] = jnp.dot(x_ref[...], w1_ref[...],
                         preferred_element_type=jnp.float32)


def _tc1b_body(h_ref, deg_ref, hs0_ref, hs1_ref, dis_ref):
    d2 = deg_ref[...]
    dis = lax.rsqrt(d2[0] + d2[1] + 1.0)
    hs = h_ref[...] * dis
    hs0_ref[...] = hs[:, :HALF]
    hs1_ref[...] = hs[:, HALF:]
    dis_ref[...] = dis


def _tc2_body(agg0_ref, agg1_ref, hs0_ref, hs1_ref, dis_ref,
              b1a_ref, b1b_ref, w2a_ref, w2b_ref, h2s_ref):
    dis = dis_ref[...]
    o0 = jnp.maximum(dis * (agg0_ref[...][0] + hs0_ref[...]) + b1a_ref[...], 0.0)
    o1 = jnp.maximum(dis * (agg1_ref[...][0] + hs1_ref[...]) + b1b_ref[...], 0.0)
    h2 = jnp.sum(o0 * w2a_ref[...] + o1 * w2b_ref[...], axis=1, keepdims=True)
    h2s_ref[...] = h2 * dis


def _tc3_body(agg2_ref, h2s_ref, dis_ref, b2_ref, out_ref):
    a = agg2_ref[...]
    out_ref[...] = dis_ref[...] * (a[0] + a[1] + h2s_ref[...]) + b2_ref[...]


RB = 1000


def kernel(x, edge_index, W1, b1, W2, b2):
    ei = edge_index.astype(jnp.int32)
    src_p = jnp.concatenate(
        [ei[0], jnp.zeros((EPAD - E,), jnp.int32)]).reshape(NBLK, BLK)
    dst_p = jnp.concatenate(
        [ei[1], jnp.full((EPAD - E,), N, jnp.int32)]).reshape(NBLK, BLK)
    zeros_1d = jnp.zeros((NPAD,), jnp.float32)
    ones_blk = jnp.ones((BLK,), jnp.float32)
    zeros_2d = jnp.zeros((BLK, HALF), jnp.float32)

    deg2 = _sc_deg(dst_p, zeros_1d, ones_blk)

    grid = (N // RB,)
    h = pl.pallas_call(
        _tc1a_body,
        grid=grid,
        in_specs=[
            pl.BlockSpec((RB, D), lambda i: (i, 0)),
            pl.BlockSpec((D, D), lambda i: (0, 0)),
        ],
        out_specs=pl.BlockSpec((RB, D), lambda i: (i, 0)),
        out_shape=jax.ShapeDtypeStruct((N, D), jnp.float32),
    )(x, W1)

    hs0, hs1, dis = pl.pallas_call(
        _tc1b_body,
        grid=grid,
        in_specs=[
            pl.BlockSpec((RB, D), lambda i: (i, 0)),
            pl.BlockSpec((2, RB, 1), lambda i: (0, i, 0)),
        ],
        out_specs=[
            pl.BlockSpec((RB, HALF), lambda i: (i, 0)),
            pl.BlockSpec((RB, HALF), lambda i: (i, 0)),
            pl.BlockSpec((RB, 1), lambda i: (i, 0)),
        ],
        out_shape=[
            jax.ShapeDtypeStruct((N, HALF), jnp.float32),
            jax.ShapeDtypeStruct((N, HALF), jnp.float32),
            jax.ShapeDtypeStruct((N, 1), jnp.float32),
        ],
    )(h, deg2.reshape(2, NPAD, 1))

    agg = _sc_agg(hs0, hs1, src_p, dst_p, zeros_2d)

    b1r = b1.reshape(1, D)
    w2r = W2.reshape(1, D)
    h2s = pl.pallas_call(
        _tc2_body,
        grid=grid,
        in_specs=[
            pl.BlockSpec((1, RB, HALF), lambda i: (0, i, 0)),
            pl.BlockSpec((1, RB, HALF), lambda i: (1, i, 0)),
            pl.BlockSpec((RB, HALF), lambda i: (i, 0)),
            pl.BlockSpec((RB, HALF), lambda i: (i, 0)),
            pl.BlockSpec((RB, 1), lambda i: (i, 0)),
            pl.BlockSpec((1, HALF), lambda i: (0, 0)),
            pl.BlockSpec((1, HALF), lambda i: (0, 1)),
            pl.BlockSpec((1, HALF), lambda i: (0, 0)),
            pl.BlockSpec((1, HALF), lambda i: (0, 1)),
        ],
        out_specs=pl.BlockSpec((RB, 1), lambda i: (i, 0)),
        out_shape=jax.ShapeDtypeStruct((N, 1), jnp.float32),
    )(agg, agg, hs0, hs1, dis, b1r, b1r, w2r, w2r)

    agg2 = _sc_agg2(h2s.reshape(N), src_p, dst_p, zeros_1d)

    out = pl.pallas_call(
        _tc3_body,
        grid=grid,
        in_specs=[
            pl.BlockSpec((2, RB, 1), lambda i: (0, i, 0)),
            pl.BlockSpec((RB, 1), lambda i: (i, 0)),
            pl.BlockSpec((RB, 1), lambda i: (i, 0)),
            pl.BlockSpec((1, 1), lambda i: (0, 0)),
        ],
        out_specs=pl.BlockSpec((RB, 1), lambda i: (i, 0)),
        out_shape=jax.ShapeDtypeStruct((N, 1), jnp.float32),
    )(agg2.reshape(2, NPAD, 1), h2s, dis, b2.reshape(1, 1))
    return out


# speedup vs baseline: 1.1314x; 1.0030x over previous
import functools

import jax
import jax.numpy as jnp
from jax import lax
from jax.experimental import pallas as pl
from jax.experimental.pallas import tpu as pltpu
from jax.experimental.pallas import tpu_sc as plsc

N = 10000
E = 160000
D = 256
HALF = 128
NPAD = 10240
EPAD = 163840
BLK = 128
NBLK = EPAD // BLK
ROWS_PER_TILE = NPAD // 16

_mesh = plsc.VectorSubcoreMesh(core_axis_name="c", subcore_axis_name="s")


@functools.partial(
    pl.kernel, mesh=_mesh,
    out_type=jax.ShapeDtypeStruct((2, NPAD), jnp.float32),
    scratch_types=[
        pltpu.VMEM((BLK,), jnp.float32),
        pltpu.VMEM((NBLK // 32, BLK), jnp.int32),
        pltpu.VMEM_SHARED((NPAD,), jnp.float32),
    ],
)
def _sc_deg(dst_hbm, zeros_hbm, ones_hbm, out_hbm, ones_v, dst_all, acc):
    c = lax.axis_index("c")
    s = lax.axis_index("s")
    r0 = pl.multiple_of(s * ROWS_PER_TILE, ROWS_PER_TILE)
    pltpu.sync_copy(zeros_hbm.at[pl.ds(r0, ROWS_PER_TILE)],
                    acc.at[pl.ds(r0, ROWS_PER_TILE)])
    pltpu.sync_copy(ones_hbm, ones_v)
    w = c * 16 + s
    nb = NBLK // 32
    pltpu.sync_copy(dst_hbm.at[pl.ds(w * nb, nb)], dst_all)
    plsc.subcore_barrier()

    def body(i, carry):
        pltpu.sync_copy(ones_v, acc.at[dst_all.at[i]], add=True)
        return carry

    lax.fori_loop(0, nb, body, 0)
    plsc.subcore_barrier()
    pltpu.sync_copy(acc.at[pl.ds(r0, ROWS_PER_TILE)],
                    out_hbm.at[c, pl.ds(r0, ROWS_PER_TILE)])


NRING = 8
NRING_AGG = 2
HB = NBLK // 32

@functools.partial(
    pl.kernel, mesh=_mesh,
    out_type=jax.ShapeDtypeStruct((2, NPAD, HALF), jnp.float32),
    scratch_types=[
        pltpu.VMEM((HB, BLK), jnp.int32),
        pltpu.VMEM((HB, BLK), jnp.int32),
    ] + [pltpu.VMEM((BLK, HALF), jnp.float32) for _ in range(NRING_AGG)]
      + [pltpu.VMEM_SHARED((NPAD, HALF), jnp.float32)]
      + [pltpu.SemaphoreType.DMA for _ in range(NRING_AGG)],
)
def _sc_agg(hs0_hbm, hs1_hbm, src_hbm, dst_hbm, z2d_hbm, out_hbm,
            src_all, dst_all, r0_v, r1_v, acc, sem0, sem1):
    rows = (r0_v, r1_v)
    sems = (sem0, sem1)
    c = lax.axis_index("c")
    s = lax.axis_index("s")
    for k in range(ROWS_PER_TILE // BLK):
        r = pl.multiple_of(s * ROWS_PER_TILE + k * BLK, BLK)
        pltpu.sync_copy(z2d_hbm, acc.at[pl.ds(r, BLK)])
    plsc.subcore_barrier()

    def run(table):
        for h in range(2):
            base = pl.multiple_of(s * (2 * HB) + h * HB, HB)
            pltpu.sync_copy(src_hbm.at[pl.ds(base, HB)], src_all)
            pltpu.sync_copy(dst_hbm.at[pl.ds(base, HB)], dst_all)
            for b in range(NRING_AGG):
                pltpu.async_copy(table.at[src_all.at[b]], rows[b], sems[b])

            def body(k, carry):
                for b in range(NRING_AGG):
                    j = k * NRING_AGG + b
                    pltpu.make_async_copy(
                        table.at[src_all.at[b]], rows[b], sems[b]).wait()
                    pltpu.sync_copy(rows[b], acc.at[dst_all.at[j]], add=True)

                    @pl.when(j + NRING_AGG < HB)
                    def _():
                        pltpu.async_copy(
                            table.at[src_all.at[j + NRING_AGG]],
                            rows[b], sems[b])
                return carry

            lax.fori_loop(0, HB // NRING_AGG, body, 0)

    @pl.when(c == 0)
    def _():
        run(hs0_hbm)

    @pl.when(c == 1)
    def _():
        run(hs1_hbm)

    plsc.subcore_barrier()
    for k in range(ROWS_PER_TILE // BLK):
        r = pl.multiple_of(s * ROWS_PER_TILE + k * BLK, BLK)
        pltpu.sync_copy(acc.at[pl.ds(r, BLK)], out_hbm.at[c, pl.ds(r, BLK)])


@functools.partial(
    pl.kernel, mesh=_mesh,
    out_type=jax.ShapeDtypeStruct((2, NPAD), jnp.float32),
    scratch_types=[
        pltpu.VMEM((NBLK // 32, BLK), jnp.int32),
        pltpu.VMEM((NBLK // 32, BLK), jnp.int32),
    ] + [pltpu.VMEM((BLK,), jnp.float32) for _ in range(NRING)]
      + [pltpu.VMEM_SHARED((NPAD,), jnp.float32)]
      + [pltpu.SemaphoreType.DMA for _ in range(NRING)],
)
def _sc_agg2(h2s_hbm, src_hbm, dst_hbm, zeros_hbm, out_hbm,
             src_all, dst_all,
             m0_v, m1_v, m2_v, m3_v, m4_v, m5_v, m6_v, m7_v, acc,
             sem0, sem1, sem2, sem3, sem4, sem5, sem6, sem7):
    msgs = (m0_v, m1_v, m2_v, m3_v, m4_v, m5_v, m6_v, m7_v)
    sems = (sem0, sem1, sem2, sem3, sem4, sem5, sem6, sem7)
    c = lax.axis_index("c")
    s = lax.axis_index("s")
    r0 = pl.multiple_of(s * ROWS_PER_TILE, ROWS_PER_TILE)
    pltpu.sync_copy(zeros_hbm.at[pl.ds(r0, ROWS_PER_TILE)],
                    acc.at[pl.ds(r0, ROWS_PER_TILE)])
    w = c * 16 + s
    nb = NBLK // 32
    pltpu.sync_copy(src_hbm.at[pl.ds(w * nb, nb)], src_all)
    pltpu.sync_copy(dst_hbm.at[pl.ds(w * nb, nb)], dst_all)
    plsc.subcore_barrier()
    for b in range(NRING):
        pltpu.async_copy(h2s_hbm.at[src_all.at[b]], msgs[b], sems[b])

    def body(k, carry):
        for b in range(NRING):
            blk = k * NRING + b
            pltpu.make_async_copy(
                h2s_hbm.at[src_all.at[b]], msgs[b], sems[b]).wait()
            pltpu.sync_copy(msgs[b], acc.at[dst_all.at[blk]], add=True)

            @pl.when(blk + NRING < nb)
            def _():
                pltpu.async_copy(
                    h2s_hbm.at[src_all.at[blk + NRING]], msgs[b], sems[b])
        return carry

    lax.fori_loop(0, nb // NRING, body, 0)
    plsc.subcore_barrier()
    pltpu.sync_copy(acc.at[pl.ds(r0, ROWS_PER_TILE)],
                    out_hbm.at[c, pl.ds(r0, ROWS_PER_TILE)])


def _tc1a_body(x_ref, w1_ref, h_ref):
    h_ref[...] = jnp.dot(x_ref[...], w1_ref[...],
                         preferred_element_type=jnp.float32)


def _tc1b_body(h_ref, deg_ref, hs0_ref, hs1_ref, dis_ref):
    d2 = deg_ref[...]
    dis = lax.rsqrt(d2[0] + d2[1] + 1.0)
    hs = h_ref[...] * dis
    hs0_ref[...] = hs[:, :HALF]
    hs1_ref[...] = hs[:, HALF:]
    dis_ref[...] = dis


def _tc2_body(agg0_ref, agg1_ref, hs0_ref, hs1_ref, dis_ref,
              b1a_ref, b1b_ref, w2a_ref, w2b_ref, h2s_ref):
    dis = dis_ref[...]
    o0 = jnp.maximum(dis * (agg0_ref[...][0] + hs0_ref[...]) + b1a_ref[...], 0.0)
    o1 = jnp.maximum(dis * (agg1_ref[...][0] + hs1_ref[...]) + b1b_ref[...], 0.0)
    h2 = jnp.sum(o0 * w2a_ref[...] + o1 * w2b_ref[...], axis=1, keepdims=True)
    h2s_ref[...] = h2 * dis


def _tc3_body(agg2_ref, h2s_ref, dis_ref, b2_ref, out_ref):
    a = agg2_ref[...]
    out_ref[...] = dis_ref[...] * (a[0] + a[1] + h2s_ref[...]) + b2_ref[...]


RB = 1000


def kernel(x, edge_index, W1, b1, W2, b2):
    ei = edge_index.astype(jnp.int32)
    src_p = jnp.concatenate(
        [ei[0], jnp.zeros((EPAD - E,), jnp.int32)]).reshape(NBLK, BLK)
    dst_p = jnp.concatenate(
        [ei[1], jnp.full((EPAD - E,), N, jnp.int32)]).reshape(NBLK, BLK)
    zeros_1d = jnp.zeros((NPAD,), jnp.float32)
    ones_blk = jnp.ones((BLK,), jnp.float32)
    zeros_2d = jnp.zeros((BLK, HALF), jnp.float32)

    deg2 = _sc_deg(dst_p, zeros_1d, ones_blk)

    grid = (N // RB,)
    h = pl.pallas_call(
        _tc1a_body,
        grid=grid,
        in_specs=[
            pl.BlockSpec((RB, D), lambda i: (i, 0)),
            pl.BlockSpec((D, D), lambda i: (0, 0)),
        ],
        out_specs=pl.BlockSpec((RB, D), lambda i: (i, 0)),
        out_shape=jax.ShapeDtypeStruct((N, D), jnp.float32),
    )(x, W1)

    hs0, hs1, dis = pl.pallas_call(
        _tc1b_body,
        grid=grid,
        in_specs=[
            pl.BlockSpec((RB, D), lambda i: (i, 0)),
            pl.BlockSpec((2, RB, 1), lambda i: (0, i, 0)),
        ],
        out_specs=[
            pl.BlockSpec((RB, HALF), lambda i: (i, 0)),
            pl.BlockSpec((RB, HALF), lambda i: (i, 0)),
            pl.BlockSpec((RB, 1), lambda i: (i, 0)),
        ],
        out_shape=[
            jax.ShapeDtypeStruct((N, HALF), jnp.float32),
            jax.ShapeDtypeStruct((N, HALF), jnp.float32),
            jax.ShapeDtypeStruct((N, 1), jnp.float32),
        ],
    )(h, deg2.reshape(2, NPAD, 1))

    agg = _sc_agg(hs0, hs1, src_p, dst_p, zeros_2d)

    b1r = b1.reshape(1, D)
    w2r = W2.reshape(1, D)
    h2s = pl.pallas_call(
        _tc2_body,
        grid=grid,
        in_specs=[
            pl.BlockSpec((1, RB, HALF), lambda i: (0, i, 0)),
            pl.BlockSpec((1, RB, HALF), lambda i: (1, i, 0)),
            pl.BlockSpec((RB, HALF), lambda i: (i, 0)),
            pl.BlockSpec((RB, HALF), lambda i: (i, 0)),
            pl.BlockSpec((RB, 1), lambda i: (i, 0)),
            pl.BlockSpec((1, HALF), lambda i: (0, 0)),
            pl.BlockSpec((1, HALF), lambda i: (0, 1)),
            pl.BlockSpec((1, HALF), lambda i: (0, 0)),
            pl.BlockSpec((1, HALF), lambda i: (0, 1)),
        ],
        out_specs=pl.BlockSpec((RB, 1), lambda i: (i, 0)),
        out_shape=jax.ShapeDtypeStruct((N, 1), jnp.float32),
    )(agg, agg, hs0, hs1, dis, b1r, b1r, w2r, w2r)

    agg2 = _sc_agg2(h2s.reshape(N), src_p, dst_p, zeros_1d)

    out = pl.pallas_call(
        _tc3_body,
        grid=grid,
        in_specs=[
            pl.BlockSpec((2, RB, 1), lambda i: (0, i, 0)),
            pl.BlockSpec((RB, 1), lambda i: (i, 0)),
            pl.BlockSpec((RB, 1), lambda i: (i, 0)),
            pl.BlockSpec((1, 1), lambda i: (0, 0)),
        ],
        out_specs=pl.BlockSpec((RB, 1), lambda i: (i, 0)),
        out_shape=jax.ShapeDtypeStruct((N, 1), jnp.float32),
    )(agg2.reshape(2, NPAD, 1), h2s, dis, b2.reshape(1, 1))
    return out


# speedup vs baseline: 1.1737x; 1.0374x over previous
import functools

import jax
import jax.numpy as jnp
from jax import lax
from jax.experimental import pallas as pl
from jax.experimental.pallas import tpu as pltpu
from jax.experimental.pallas import tpu_sc as plsc

N = 10000
E = 160000
D = 256
HALF = 128
NPAD = 10240
EPAD = 163840
BLK = 128
NBLK = EPAD // BLK
ROWS_PER_TILE = NPAD // 16

_mesh = plsc.VectorSubcoreMesh(core_axis_name="c", subcore_axis_name="s")


@functools.partial(
    pl.kernel, mesh=_mesh,
    out_type=jax.ShapeDtypeStruct((2, NPAD), jnp.float32),
    scratch_types=[
        pltpu.VMEM((BLK,), jnp.float32),
        pltpu.VMEM((NBLK // 32, BLK), jnp.int32),
        pltpu.VMEM_SHARED((NPAD,), jnp.float32),
    ],
)
def _sc_deg(dst_hbm, zeros_hbm, ones_hbm, out_hbm, ones_v, dst_all, acc):
    c = lax.axis_index("c")
    s = lax.axis_index("s")
    r0 = pl.multiple_of(s * ROWS_PER_TILE, ROWS_PER_TILE)
    pltpu.sync_copy(zeros_hbm.at[pl.ds(r0, ROWS_PER_TILE)],
                    acc.at[pl.ds(r0, ROWS_PER_TILE)])
    pltpu.sync_copy(ones_hbm, ones_v)
    w = c * 16 + s
    nb = NBLK // 32
    pltpu.sync_copy(dst_hbm.at[pl.ds(w * nb, nb)], dst_all)
    plsc.subcore_barrier()

    def body(i, carry):
        pltpu.sync_copy(ones_v, acc.at[dst_all.at[i]], add=True)
        return carry

    lax.fori_loop(0, nb, body, 0)
    plsc.subcore_barrier()
    pltpu.sync_copy(acc.at[pl.ds(r0, ROWS_PER_TILE)],
                    out_hbm.at[c, pl.ds(r0, ROWS_PER_TILE)])


NRING = 8
NRING_AGG = 2
HB = NBLK // 32

@functools.partial(
    pl.kernel, mesh=_mesh,
    out_type=jax.ShapeDtypeStruct((2, NPAD, HALF), jnp.float32),
    scratch_types=[
        pltpu.VMEM((HB, BLK), jnp.int32),
        pltpu.VMEM((HB, BLK), jnp.int32),
    ] + [pltpu.VMEM((BLK, HALF), jnp.float32) for _ in range(NRING_AGG)]
      + [pltpu.VMEM_SHARED((NPAD, HALF), jnp.float32)]
      + [pltpu.SemaphoreType.DMA for _ in range(NRING_AGG)],
)
def _sc_agg(hs0_hbm, hs1_hbm, src_hbm, dst_hbm, out_hbm,
            src_all, dst_all, r0_v, r1_v, acc, sem0, sem1):
    rows = (r0_v, r1_v)
    sems = (sem0, sem1)
    c = lax.axis_index("c")
    s = lax.axis_index("s")

    def run(table):
        for k in range(ROWS_PER_TILE // BLK):
            r = pl.multiple_of(s * ROWS_PER_TILE + k * BLK, BLK)
            pltpu.sync_copy(table.at[pl.ds(r, BLK)], acc.at[pl.ds(r, BLK)])
        plsc.subcore_barrier()
        for h in range(2):
            base = pl.multiple_of(s * (2 * HB) + h * HB, HB)
            pltpu.sync_copy(src_hbm.at[pl.ds(base, HB)], src_all)
            pltpu.sync_copy(dst_hbm.at[pl.ds(base, HB)], dst_all)
            for b in range(NRING_AGG):
                pltpu.async_copy(table.at[src_all.at[b]], rows[b], sems[b])

            def body(k, carry):
                for b in range(NRING_AGG):
                    j = k * NRING_AGG + b
                    pltpu.make_async_copy(
                        table.at[src_all.at[b]], rows[b], sems[b]).wait()
                    pltpu.sync_copy(rows[b], acc.at[dst_all.at[j]], add=True)

                    @pl.when(j + NRING_AGG < HB)
                    def _():
                        pltpu.async_copy(
                            table.at[src_all.at[j + NRING_AGG]],
                            rows[b], sems[b])
                return carry

            lax.fori_loop(0, HB // NRING_AGG, body, 0)

    @pl.when(c == 0)
    def _():
        run(hs0_hbm)

    @pl.when(c == 1)
    def _():
        run(hs1_hbm)

    plsc.subcore_barrier()
    for k in range(ROWS_PER_TILE // BLK):
        r = pl.multiple_of(s * ROWS_PER_TILE + k * BLK, BLK)
        pltpu.sync_copy(acc.at[pl.ds(r, BLK)], out_hbm.at[c, pl.ds(r, BLK)])


@functools.partial(
    pl.kernel, mesh=_mesh,
    out_type=jax.ShapeDtypeStruct((2, NPAD), jnp.float32),
    scratch_types=[
        pltpu.VMEM((NBLK // 32, BLK), jnp.int32),
        pltpu.VMEM((NBLK // 32, BLK), jnp.int32),
    ] + [pltpu.VMEM((BLK,), jnp.float32) for _ in range(NRING)]
      + [pltpu.VMEM_SHARED((NPAD,), jnp.float32)]
      + [pltpu.SemaphoreType.DMA for _ in range(NRING)],
)
def _sc_agg2(h2s_hbm, src_hbm, dst_hbm, zeros_hbm, out_hbm,
             src_all, dst_all,
             m0_v, m1_v, m2_v, m3_v, m4_v, m5_v, m6_v, m7_v, acc,
             sem0, sem1, sem2, sem3, sem4, sem5, sem6, sem7):
    msgs = (m0_v, m1_v, m2_v, m3_v, m4_v, m5_v, m6_v, m7_v)
    sems = (sem0, sem1, sem2, sem3, sem4, sem5, sem6, sem7)
    c = lax.axis_index("c")
    s = lax.axis_index("s")
    r0 = pl.multiple_of(s * ROWS_PER_TILE, ROWS_PER_TILE)
    pltpu.sync_copy(zeros_hbm.at[pl.ds(r0, ROWS_PER_TILE)],
                    acc.at[pl.ds(r0, ROWS_PER_TILE)])
    w = c * 16 + s
    nb = NBLK // 32
    pltpu.sync_copy(src_hbm.at[pl.ds(w * nb, nb)], src_all)
    pltpu.sync_copy(dst_hbm.at[pl.ds(w * nb, nb)], dst_all)
    plsc.subcore_barrier()
    for b in range(NRING):
        pltpu.async_copy(h2s_hbm.at[src_all.at[b]], msgs[b], sems[b])

    def body(k, carry):
        for b in range(NRING):
            blk = k * NRING + b
            pltpu.make_async_copy(
                h2s_hbm.at[src_all.at[b]], msgs[b], sems[b]).wait()
            pltpu.sync_copy(msgs[b], acc.at[dst_all.at[blk]], add=True)

            @pl.when(blk + NRING < nb)
            def _():
                pltpu.async_copy(
                    h2s_hbm.at[src_all.at[blk + NRING]], msgs[b], sems[b])
        return carry

    lax.fori_loop(0, nb // NRING, body, 0)
    plsc.subcore_barrier()
    pltpu.sync_copy(acc.at[pl.ds(r0, ROWS_PER_TILE)],
                    out_hbm.at[c, pl.ds(r0, ROWS_PER_TILE)])


def _tc1a_body(x_ref, w1_ref, h_ref):
    h_ref[...] = jnp.dot(x_ref[...], w1_ref[...],
                         preferred_element_type=jnp.float32)


def _tc1b_body(h_ref, deg_ref, hs0_ref, hs1_ref, dis_ref):
    d2 = deg_ref[...]
    dis = lax.rsqrt(d2[0] + d2[1] + 1.0)
    hs = h_ref[...] * dis
    hs0_ref[...] = hs[:, :HALF]
    hs1_ref[...] = hs[:, HALF:]
    dis_ref[...] = dis


def _tc2_body(agg0_ref, agg1_ref, dis_ref,
              b1a_ref, b1b_ref, w2a_ref, w2b_ref, h2s_ref):
    dis = dis_ref[...]
    o0 = jnp.maximum(dis * agg0_ref[...][0] + b1a_ref[...], 0.0)
    o1 = jnp.maximum(dis * agg1_ref[...][0] + b1b_ref[...], 0.0)
    h2 = jnp.sum(o0 * w2a_ref[...] + o1 * w2b_ref[...], axis=1, keepdims=True)
    h2s_ref[...] = h2 * dis


def _tc3_body(agg2_ref, h2s_ref, dis_ref, b2_ref, out_ref):
    a = agg2_ref[...]
    out_ref[...] = dis_ref[...] * (a[0] + a[1] + h2s_ref[...]) + b2_ref[...]


RB = 1000


def kernel(x, edge_index, W1, b1, W2, b2):
    ei = edge_index.astype(jnp.int32)
    src_p = jnp.concatenate(
        [ei[0], jnp.zeros((EPAD - E,), jnp.int32)]).reshape(NBLK, BLK)
    dst_p = jnp.concatenate(
        [ei[1], jnp.full((EPAD - E,), N, jnp.int32)]).reshape(NBLK, BLK)
    zeros_1d = jnp.zeros((NPAD,), jnp.float32)
    ones_blk = jnp.ones((BLK,), jnp.float32)

    deg2 = _sc_deg(dst_p, zeros_1d, ones_blk)

    grid = (N // RB,)
    h = pl.pallas_call(
        _tc1a_body,
        grid=grid,
        in_specs=[
            pl.BlockSpec((RB, D), lambda i: (i, 0)),
            pl.BlockSpec((D, D), lambda i: (0, 0)),
        ],
        out_specs=pl.BlockSpec((RB, D), lambda i: (i, 0)),
        out_shape=jax.ShapeDtypeStruct((N, D), jnp.float32),
    )(x, W1)

    hs0, hs1, dis = pl.pallas_call(
        _tc1b_body,
        grid=grid,
        in_specs=[
            pl.BlockSpec((RB, D), lambda i: (i, 0)),
            pl.BlockSpec((2, RB, 1), lambda i: (0, i, 0)),
        ],
        out_specs=[
            pl.BlockSpec((RB, HALF), lambda i: (i, 0)),
            pl.BlockSpec((RB, HALF), lambda i: (i, 0)),
            pl.BlockSpec((RB, 1), lambda i: (i, 0)),
        ],
        out_shape=[
            jax.ShapeDtypeStruct((NPAD, HALF), jnp.float32),
            jax.ShapeDtypeStruct((NPAD, HALF), jnp.float32),
            jax.ShapeDtypeStruct((N, 1), jnp.float32),
        ],
    )(h, deg2.reshape(2, NPAD, 1))

    agg = _sc_agg(hs0, hs1, src_p, dst_p)

    b1r = b1.reshape(1, D)
    w2r = W2.reshape(1, D)
    h2s = pl.pallas_call(
        _tc2_body,
        grid=grid,
        in_specs=[
            pl.BlockSpec((1, RB, HALF), lambda i: (0, i, 0)),
            pl.BlockSpec((1, RB, HALF), lambda i: (1, i, 0)),
            pl.BlockSpec((RB, 1), lambda i: (i, 0)),
            pl.BlockSpec((1, HALF), lambda i: (0, 0)),
            pl.BlockSpec((1, HALF), lambda i: (0, 1)),
            pl.BlockSpec((1, HALF), lambda i: (0, 0)),
            pl.BlockSpec((1, HALF), lambda i: (0, 1)),
        ],
        out_specs=pl.BlockSpec((RB, 1), lambda i: (i, 0)),
        out_shape=jax.ShapeDtypeStruct((N, 1), jnp.float32),
    )(agg, agg, dis, b1r, b1r, w2r, w2r)

    agg2 = _sc_agg2(h2s.reshape(N), src_p, dst_p, zeros_1d)

    out = pl.pallas_call(
        _tc3_body,
        grid=grid,
        in_specs=[
            pl.BlockSpec((2, RB, 1), lambda i: (0, i, 0)),
            pl.BlockSpec((RB, 1), lambda i: (i, 0)),
            pl.BlockSpec((RB, 1), lambda i: (i, 0)),
            pl.BlockSpec((1, 1), lambda i: (0, 0)),
        ],
        out_specs=pl.BlockSpec((RB, 1), lambda i: (i, 0)),
        out_shape=jax.ShapeDtypeStruct((N, 1), jnp.float32),
    )(agg2.reshape(2, NPAD, 1), h2s, dis, b2.reshape(1, 1))
    return out
